# baseline (speedup 1.0000x reference)
import functools

import jax
import jax.numpy as jnp
from jax import lax
from jax.experimental import pallas as pl
from jax.experimental.pallas import tpu as pltpu
from jax.experimental.pallas import tpu_sc as plsc

V = 100000
C = 128
D = 128
T = 50
L = 200
B = 1024

NC = 2
NS = 16
LANES = 16
NW = NC * NS
B_PER_W = B // NW
HALF_L = L // 2
CV = C // LANES


def _sc_pool(xr, table):
  mesh = plsc.VectorSubcoreMesh(core_axis_name="c", subcore_axis_name="s")

  @functools.partial(
      pl.kernel,
      mesh=mesh,
      out_type=jax.ShapeDtypeStruct((B, C), jnp.float32),
      scratch_types=[
          pltpu.VMEM((2 * B_PER_W, HALF_L), jnp.int32),
          pltpu.VMEM((4, L, C), jnp.float32),
          pltpu.VMEM((B_PER_W, C), jnp.float32),
          pltpu.SemaphoreType.DMA,
          pltpu.SemaphoreType.DMA,
          pltpu.SemaphoreType.DMA,
          pltpu.SemaphoreType.DMA,
      ],
  )
  def k(xr_hbm, table_hbm, out_hbm, idx_v, rows_v, out_v, s0, s1, s2, s3):
    wid = lax.axis_index("s") * NC + lax.axis_index("c")
    base = wid * B_PER_W
    pltpu.sync_copy(xr_hbm.at[pl.ds(2 * base, 2 * B_PER_W)], idx_v)
    sems = (s0, s1, s2, s3)

    def fire(j, b):
      pltpu.async_copy(
          table_hbm.at[idx_v.at[2 * j]], rows_v.at[b, pl.ds(0, HALF_L)],
          sems[b])
      pltpu.async_copy(
          table_hbm.at[idx_v.at[2 * j + 1]],
          rows_v.at[b, pl.ds(HALF_L, HALF_L)], sems[b])

    def drain(b):
      pltpu.make_async_copy(
          table_hbm.at[pl.ds(0, L)], rows_v.at[b], sems[b]).wait()

    fire(0, 0)
    fire(1, 1)
    fire(2, 2)
    zero = jnp.zeros((LANES,), jnp.float32)

    def body(i, _):
      for b in range(4):
        j = 4 * i + b
        nxt = j + 3

        @pl.when(nxt < B_PER_W)
        def _():
          fire(nxt, (b + 3) % 4)

        drain(b)

        def acc_body(l, carry):
          return tuple(
              carry[c] + rows_v[b, l, pl.ds(c * LANES, LANES)]
              for c in range(CV))

        acc = lax.fori_loop(0, L, acc_body, (zero,) * CV)
        for c in range(CV):
          out_v[j, pl.ds(c * LANES, LANES)] = acc[c]
      return 0

    lax.fori_loop(0, B_PER_W // 4, body, 0)
    pltpu.sync_copy(out_v, out_hbm.at[pl.ds(base, B_PER_W)])

  return k(xr, table)


def _ffn_body(feat_ref, w1_ref, b1_ref, wt_ref, bt_ref, logit_ref, topic_ref):
  feat = feat_ref[...]
  logit_ref[...] = lax.dot_general(
      feat, w1_ref[...], (((1,), (1,)), ((), ())),
      preferred_element_type=jnp.float32) + b1_ref[...]
  tl = lax.dot_general(
      feat, wt_ref[...], (((1,), (1,)), ((), ())),
      preferred_element_type=jnp.float32) + bt_ref[...]
  m = jnp.max(tl, axis=1, keepdims=True)
  e = jnp.exp(tl - m)
  s = jnp.sum(e, axis=1, keepdims=True)
  topic_ref[...] = tl - m - jnp.log(s)


def _tc_ffn(feat, fc1_W, fc1_b, topic_W, topic_b):
  blk = 1024
  grid = B // blk
  return pl.pallas_call(
      _ffn_body,
      grid=(grid,),
      in_specs=[
          pl.BlockSpec((blk, C), lambda i: (i, 0)),
          pl.BlockSpec((C, D), lambda i: (0, 0)),
          pl.BlockSpec((C,), lambda i: (0,)),
          pl.BlockSpec((T, D), lambda i: (0, 0)),
          pl.BlockSpec((T,), lambda i: (0,)),
      ],
      out_specs=[
          pl.BlockSpec((blk, C), lambda i: (i, 0)),
          pl.BlockSpec((blk, T), lambda i: (i, 0)),
      ],
      out_shape=[
          jax.ShapeDtypeStruct((B, C), jnp.float32),
          jax.ShapeDtypeStruct((B, T), jnp.float32),
      ],
  )(feat, fc1_W, fc1_b, topic_W, topic_b)


def kernel(x, embed_table, fc1_W, fc1_b, topic_W, topic_b):
  xr = jnp.transpose(x, (1, 0)).reshape(2 * B, HALF_L)
  feat = _sc_pool(xr, embed_table)
  return _tc_ffn(feat, fc1_W, fc1_b, topic_W, topic_b)


# speedup vs baseline: 1.0034x; 1.0034x over previous
import functools

import jax
import jax.numpy as jnp
from jax import lax
from jax.experimental import pallas as pl
from jax.experimental.pallas import tpu as pltpu
from jax.experimental.pallas import tpu_sc as plsc

V = 100000
C = 128
D = 128
T = 50
L = 200
B = 1024

NC = 2
NS = 16
LANES = 16
NW = NC * NS
B_PER_W = B // NW
HALF_L = L // 2
CV = C // LANES


def _sc_pool(xr, table):
  mesh = plsc.VectorSubcoreMesh(core_axis_name="c", subcore_axis_name="s")

  @functools.partial(
      pl.kernel,
      mesh=mesh,
      out_type=jax.ShapeDtypeStruct((B, C), jnp.float32),
      scratch_types=[
          pltpu.VMEM((2 * B_PER_W, HALF_L), jnp.int32),
          pltpu.VMEM((4, L, C), jnp.float32),
          pltpu.VMEM((B_PER_W, C), jnp.float32),
          pltpu.SemaphoreType.DMA,
          pltpu.SemaphoreType.DMA,
          pltpu.SemaphoreType.DMA,
          pltpu.SemaphoreType.DMA,
      ],
  )
  def k(xr_hbm, table_hbm, out_hbm, idx_v, rows_v, out_v, s0, s1, s2, s3):
    wid = lax.axis_index("s") * NC + lax.axis_index("c")
    base = wid * B_PER_W
    pltpu.sync_copy(xr_hbm.at[pl.ds(2 * base, 2 * B_PER_W)], idx_v)
    sems = (s0, s1, s2, s3)

    def fire(j, b):
      pltpu.async_copy(
          table_hbm.at[idx_v.at[2 * j]], rows_v.at[b, pl.ds(0, HALF_L)],
          sems[b])
      pltpu.async_copy(
          table_hbm.at[idx_v.at[2 * j + 1]],
          rows_v.at[b, pl.ds(HALF_L, HALF_L)], sems[b])

    def drain(b):
      pltpu.make_async_copy(
          table_hbm.at[pl.ds(0, L)], rows_v.at[b], sems[b]).wait()

    fire(0, 0)
    fire(1, 1)
    fire(2, 2)
    zero = jnp.zeros((LANES,), jnp.float32)

    def body(i, _):
      for b in range(4):
        j = 4 * i + b
        nxt = j + 3

        @pl.when(nxt < B_PER_W)
        def _():
          fire(nxt, (b + 3) % 4)

        drain(b)

        def acc_body(l, carry):
          return tuple(
              carry[c] + rows_v[b, l, pl.ds(c * LANES, LANES)]
              for c in range(CV))

        acc = lax.fori_loop(0, L, acc_body, (zero,) * CV, unroll=2)
        for c in range(CV):
          out_v[j, pl.ds(c * LANES, LANES)] = acc[c]
      return 0

    lax.fori_loop(0, B_PER_W // 4, body, 0)
    pltpu.sync_copy(out_v, out_hbm.at[pl.ds(base, B_PER_W)])

  return k(xr, table)


def _ffn_body(feat_ref, w1_ref, b1_ref, wt_ref, bt_ref, logit_ref, topic_ref):
  feat = feat_ref[...]
  logit_ref[...] = lax.dot_general(
      feat, w1_ref[...], (((1,), (1,)), ((), ())),
      preferred_element_type=jnp.float32) + b1_ref[...]
  tl = lax.dot_general(
      feat, wt_ref[...], (((1,), (1,)), ((), ())),
      preferred_element_type=jnp.float32) + bt_ref[...]
  m = jnp.max(tl, axis=1, keepdims=True)
  e = jnp.exp(tl - m)
  s = jnp.sum(e, axis=1, keepdims=True)
  topic_ref[...] = tl - m - jnp.log(s)


def _tc_ffn(feat, fc1_W, fc1_b, topic_W, topic_b):
  blk = 1024
  grid = B // blk
  return pl.pallas_call(
      _ffn_body,
      grid=(grid,),
      in_specs=[
          pl.BlockSpec((blk, C), lambda i: (i, 0)),
          pl.BlockSpec((C, D), lambda i: (0, 0)),
          pl.BlockSpec((C,), lambda i: (0,)),
          pl.BlockSpec((T, D), lambda i: (0, 0)),
          pl.BlockSpec((T,), lambda i: (0,)),
      ],
      out_specs=[
          pl.BlockSpec((blk, C), lambda i: (i, 0)),
          pl.BlockSpec((blk, T), lambda i: (i, 0)),
      ],
      out_shape=[
          jax.ShapeDtypeStruct((B, C), jnp.float32),
          jax.ShapeDtypeStruct((B, T), jnp.float32),
      ],
  )(feat, fc1_W, fc1_b, topic_W, topic_b)


def kernel(x, embed_table, fc1_W, fc1_b, topic_W, topic_b):
  xr = jnp.transpose(x, (1, 0)).reshape(2 * B, HALF_L)
  feat = _sc_pool(xr, embed_table)
  return _tc_ffn(feat, fc1_W, fc1_b, topic_W, topic_b)
